# Initial kernel scaffold; baseline (speedup 1.0000x reference)
import jax
import jax.numpy as jnp
from jax.experimental import pallas as pl


def kernel(z, codebook):
    raise NotImplementedError("write your pallas kernel here")


# speedup vs baseline: 1.0244x; 1.0244x over previous
import jax
import jax.numpy as jnp
from jax.experimental import pallas as pl
from jax.experimental.pallas import tpu as pltpu

_NUM_CODES = 8192
_DIM = 32
_BETA = 0.25
_B = 8192
_BB = 256
_GRID = _B // _BB
_NWIN = 4
_W = _NUM_CODES // _NWIN


def _vq_body(z_ref, cb_ref, q_ref, idx_ref, commit_ref, ent_ref, used_ref,
             counts_ref, acc_ref):
    i = pl.program_id(0)
    z = z_ref[...]
    cb = cb_ref[...]
    zb = z.astype(jnp.bfloat16)
    cbb = cb.astype(jnp.bfloat16)
    z2 = jnp.sum(z * z, axis=1, keepdims=True)
    e2 = jnp.sum(cb * cb, axis=1)[None, :]
    ze = jax.lax.dot_general(zb, cbb, (((1,), (1,)), ((), ())),
                             preferred_element_type=jnp.float32)
    dist = (z2 + e2) - 2.0 * ze

    acc_v = jnp.full((_BB, 1), jnp.inf, jnp.float32)
    acc_i = jnp.zeros((_BB, 1), jnp.int32)
    for w in range(_NWIN):
        dw = jax.lax.slice_in_dim(dist, w * _W, (w + 1) * _W, axis=1)
        wmin = jnp.min(dw, axis=1, keepdims=True)
        io = jax.lax.broadcasted_iota(jnp.int32, dw.shape, 1) + w * _W
        widx = jnp.min(jnp.where(dw == wmin, io, _NUM_CODES), axis=1,
                       keepdims=True)
        lt = wmin < acc_v
        eq = (wmin == acc_v) & (widx < acc_i)
        acc_v = jnp.where(lt, wmin, acc_v)
        acc_v = acc_v.astype(jnp.bfloat16).astype(jnp.float32)
        acc_i = jnp.where(lt | eq, widx, acc_i)
    idx = acc_i[:, 0]
    idx_ref[...] = idx

    iota = jax.lax.broadcasted_iota(jnp.int32, (_BB, _NUM_CODES), 1)
    one_hot = (iota == acc_i).astype(jnp.bfloat16)
    q = jax.lax.dot_general(one_hot, cbb, (((1,), (0,)), ((), ())),
                            preferred_element_type=jnp.float32)
    q_ref[...] = q

    block_counts = jnp.sum(one_hot.astype(jnp.float32), axis=0,
                           keepdims=True)
    block_commit = jnp.sum((z - q) ** 2)

    @pl.when(i == 0)
    def _():
        counts_ref[...] = block_counts
        acc_ref[0] = block_commit

    @pl.when(i > 0)
    def _():
        counts_ref[...] += block_counts
        acc_ref[0] += block_commit

    @pl.when(i == _GRID - 1)
    def _():
        counts = counts_ref[...]
        usage = counts * (1.0 / _B) + 1e-10
        ent_ref[...] = jnp.reshape(-jnp.sum(usage * jnp.log(usage)), (1, 1))
        used_ref[...] = jnp.reshape(
            jnp.sum((counts > 0.0).astype(jnp.int32)), (1, 1))
        commit_ref[...] = jnp.reshape(
            acc_ref[0] * (_BETA / (_B * _DIM)), (1, 1))


def kernel(z, codebook):
    q, idx, commit, ent, used = pl.pallas_call(
        _vq_body,
        grid=(_GRID,),
        in_specs=[
            pl.BlockSpec((_BB, _DIM), lambda i: (i, 0)),
            pl.BlockSpec((_NUM_CODES, _DIM), lambda i: (0, 0)),
        ],
        out_specs=[
            pl.BlockSpec((_BB, _DIM), lambda i: (i, 0)),
            pl.BlockSpec((_BB,), lambda i: (i,)),
            pl.BlockSpec((1, 1), lambda i: (0, 0)),
            pl.BlockSpec((1, 1), lambda i: (0, 0)),
            pl.BlockSpec((1, 1), lambda i: (0, 0)),
        ],
        out_shape=[
            jax.ShapeDtypeStruct((_B, _DIM), jnp.float32),
            jax.ShapeDtypeStruct((_B,), jnp.int32),
            jax.ShapeDtypeStruct((1, 1), jnp.float32),
            jax.ShapeDtypeStruct((1, 1), jnp.float32),
            jax.ShapeDtypeStruct((1, 1), jnp.int32),
        ],
        scratch_shapes=[
            pltpu.VMEM((1, _NUM_CODES), jnp.float32),
            pltpu.SMEM((1,), jnp.float32),
        ],
    )(z, codebook)
    return (q, idx, commit[0, 0], ent[0, 0], used[0, 0])


# speedup vs baseline: 1.0479x; 1.0229x over previous
import jax
import jax.numpy as jnp
from jax.experimental import pallas as pl
from jax.experimental.pallas import tpu as pltpu

_NUM_CODES = 8192
_DIM = 32
_BETA = 0.25
_B = 8192
_BB = 256
_GRID = _B // _BB
_NWIN = 4
_W = _NUM_CODES // _NWIN


def _vq_body(z_ref, zs_ref, cb_ref, cbb_ref, q_ref, idx_ref, commit_ref,
             ent_ref, used_ref, e2_ref, counts_ref, acc_ref):
    i = pl.program_id(0)
    z = z_ref[...]
    zs = zs_ref[...]
    cbb = cbb_ref[...]

    @pl.when(i == 0)
    def _():
        cb = cb_ref[...]
        e2_ref[...] = jnp.sum(cb * cb, axis=1)[None, :]

    z2 = jnp.sum(z * z, axis=1, keepdims=True)
    e2 = e2_ref[...]
    nze2 = jax.lax.dot_general(zs, cbb, (((1,), (1,)), ((), ())),
                               preferred_element_type=jnp.float32)
    dist = (z2 + e2) + nze2

    iota_k = jax.lax.broadcasted_iota(jnp.int32, (1, _NUM_CODES), 1)

    acc_v = jnp.full((_BB, 1), jnp.inf, jnp.float32)
    acc_i = jnp.zeros((_BB, 1), jnp.int32)
    for w in range(_NWIN):
        dw = jax.lax.slice_in_dim(dist, w * _W, (w + 1) * _W, axis=1)
        io = jax.lax.slice_in_dim(iota_k, w * _W, (w + 1) * _W, axis=1)
        wmin = jnp.min(dw, axis=1, keepdims=True)
        widx = jnp.min(jnp.where(dw == wmin, io, _NUM_CODES), axis=1,
                       keepdims=True)
        lt = wmin < acc_v
        eq = (wmin == acc_v) & (widx < acc_i)
        acc_v = jnp.where(lt, wmin, acc_v)
        acc_v = acc_v.astype(jnp.bfloat16).astype(jnp.float32)
        acc_i = jnp.where(lt | eq, widx, acc_i)
    idx_ref[...] = acc_i[:, 0]

    one_hot = (iota_k == acc_i).astype(jnp.bfloat16)
    q = jax.lax.dot_general(one_hot, cbb, (((1,), (0,)), ((), ())),
                            preferred_element_type=jnp.float32)
    q_ref[...] = q

    ones_row = jnp.ones((1, _BB), jnp.bfloat16)
    block_counts = jax.lax.dot_general(ones_row, one_hot,
                                       (((1,), (0,)), ((), ())),
                                       preferred_element_type=jnp.float32)
    block_commit = jnp.sum((z - q) ** 2)

    @pl.when(i == 0)
    def _():
        counts_ref[...] = block_counts
        acc_ref[0] = block_commit

    @pl.when(i > 0)
    def _():
        counts_ref[...] += block_counts
        acc_ref[0] += block_commit

    @pl.when(i == _GRID - 1)
    def _():
        counts = counts_ref[...]
        usage = counts * (1.0 / _B) + 1e-10
        ent_ref[...] = jnp.reshape(-jnp.sum(usage * jnp.log(usage)), (1, 1))
        used_ref[...] = jnp.reshape(
            jnp.sum((counts > 0.0).astype(jnp.int32)), (1, 1))
        commit_ref[...] = jnp.reshape(
            acc_ref[0] * (_BETA / (_B * _DIM)), (1, 1))


def kernel(z, codebook):
    zs = (-2.0 * z).astype(jnp.bfloat16)
    cbb = codebook.astype(jnp.bfloat16)
    q, idx, commit, ent, used = pl.pallas_call(
        _vq_body,
        grid=(_GRID,),
        in_specs=[
            pl.BlockSpec((_BB, _DIM), lambda i: (i, 0)),
            pl.BlockSpec((_BB, _DIM), lambda i: (i, 0)),
            pl.BlockSpec((_NUM_CODES, _DIM), lambda i: (0, 0)),
            pl.BlockSpec((_NUM_CODES, _DIM), lambda i: (0, 0)),
        ],
        out_specs=[
            pl.BlockSpec((_BB, _DIM), lambda i: (i, 0)),
            pl.BlockSpec((_BB,), lambda i: (i,)),
            pl.BlockSpec((1, 1), lambda i: (0, 0)),
            pl.BlockSpec((1, 1), lambda i: (0, 0)),
            pl.BlockSpec((1, 1), lambda i: (0, 0)),
        ],
        out_shape=[
            jax.ShapeDtypeStruct((_B, _DIM), jnp.float32),
            jax.ShapeDtypeStruct((_B,), jnp.int32),
            jax.ShapeDtypeStruct((1, 1), jnp.float32),
            jax.ShapeDtypeStruct((1, 1), jnp.float32),
            jax.ShapeDtypeStruct((1, 1), jnp.int32),
        ],
        scratch_shapes=[
            pltpu.VMEM((1, _NUM_CODES), jnp.float32),
            pltpu.VMEM((1, _NUM_CODES), jnp.float32),
            pltpu.SMEM((1,), jnp.float32),
        ],
    )(z, zs, codebook, cbb)
    return (q, idx, commit[0, 0], ent[0, 0], used[0, 0])


# speedup vs baseline: 1.1894x; 1.1351x over previous
import functools

import jax
import jax.numpy as jnp
from jax import lax
from jax.experimental import pallas as pl
from jax.experimental.pallas import tpu as pltpu
from jax.experimental.pallas import tpu_sc as plsc

_NUM_CODES = 8192
_DIM = 32
_BETA = 0.25
_B = 8192
_BB = 256
_GRID = _B // _BB
_NWIN = 4
_W = _NUM_CODES // _NWIN

_NW = 32
_BPW = _B // _NW
_PD = 128


def _vq_body(z_ref, zs_ref, cb_ref, cbb_ref, idx_ref, idx3_ref, commit_ref,
             e2_ref, acc_ref):
    i = pl.program_id(0)
    z = z_ref[...]
    zs = zs_ref[...]
    cbb = cbb_ref[...]

    @pl.when(i == 0)
    def _():
        cb = cb_ref[...]
        e2_ref[...] = jnp.sum(cb * cb, axis=1)[None, :]

    z2 = jnp.sum(z * z, axis=1, keepdims=True)
    e2 = e2_ref[...]
    nze2 = jax.lax.dot_general(zs, cbb, (((1,), (1,)), ((), ())),
                               preferred_element_type=jnp.float32)
    dist = (z2 + e2) + nze2

    iota_k = jax.lax.broadcasted_iota(jnp.int32, (1, _NUM_CODES), 1)

    acc_v = jnp.full((_BB, 1), jnp.inf, jnp.float32)
    acc_i = jnp.zeros((_BB, 1), jnp.int32)
    tv = jnp.full((_BB, 1), jnp.inf, jnp.float32)
    for w in range(_NWIN):
        dw = jax.lax.slice_in_dim(dist, w * _W, (w + 1) * _W, axis=1)
        io = jax.lax.slice_in_dim(iota_k, w * _W, (w + 1) * _W, axis=1)
        wmin = jnp.min(dw, axis=1, keepdims=True)
        widx = jnp.min(jnp.where(dw == wmin, io, _NUM_CODES), axis=1,
                       keepdims=True)
        lt = wmin < acc_v
        take = lt | ((wmin == acc_v) & (widx < acc_i))
        acc_v = jnp.where(lt, wmin, acc_v)
        acc_v = acc_v.astype(jnp.bfloat16).astype(jnp.float32)
        acc_i = jnp.where(take, widx, acc_i)
        tv = jnp.where(take, wmin, tv)
    idx = acc_i[:, 0]
    idx_ref[...] = idx
    idx3_ref[...] = jnp.reshape(idx, (1, _BB // 128, 128))

    block_commit = jnp.sum(tv)

    @pl.when(i == 0)
    def _():
        acc_ref[0] = block_commit

    @pl.when(i > 0)
    def _():
        acc_ref[0] += block_commit

    @pl.when(i == _GRID - 1)
    def _():
        commit_ref[...] = jnp.reshape(
            acc_ref[0] * (_BETA / (_B * _DIM)), (1, 1))


def _tc_main(z, zs, codebook, cbb):
    return pl.pallas_call(
        _vq_body,
        grid=(_GRID,),
        in_specs=[
            pl.BlockSpec((_BB, _DIM), lambda i: (i, 0)),
            pl.BlockSpec((_BB, _DIM), lambda i: (i, 0)),
            pl.BlockSpec((_NUM_CODES, _DIM), lambda i: (0, 0)),
            pl.BlockSpec((_NUM_CODES, _DIM), lambda i: (0, 0)),
        ],
        out_specs=[
            pl.BlockSpec((_BB,), lambda i: (i,)),
            pl.BlockSpec((1, _BB // 128, 128), lambda i: (i, 0, 0)),
            pl.BlockSpec((1, 1), lambda i: (0, 0)),
        ],
        out_shape=[
            jax.ShapeDtypeStruct((_B,), jnp.int32),
            jax.ShapeDtypeStruct((_GRID, _BB // 128, 128), jnp.int32),
            jax.ShapeDtypeStruct((1, 1), jnp.float32),
        ],
        scratch_shapes=[
            pltpu.VMEM((1, _NUM_CODES), jnp.float32),
            pltpu.SMEM((1,), jnp.float32),
        ],
    )(z, zs, codebook, cbb)


_N128 = _BPW // 128


@functools.partial(
    pl.kernel,
    out_type=[
        jax.ShapeDtypeStruct((_B, _PD), jnp.float32),
        jax.ShapeDtypeStruct((2, _NUM_CODES), jnp.float32),
    ],
    mesh=plsc.VectorSubcoreMesh(core_axis_name="c", subcore_axis_name="s"),
    scratch_types=[
        pltpu.VMEM((_N128, 128), jnp.int32),
        pltpu.VMEM((_BPW, _PD), jnp.float32),
        pltpu.VMEM((_NUM_CODES,), jnp.float32),
        pltpu.VMEM((128,), jnp.float32),
        pltpu.VMEM_SHARED((_NUM_CODES,), jnp.float32),
        pltpu.SemaphoreType.DMA,
    ],
)
def _sc_gather_hist(table_hbm, idx3_hbm, q_hbm, counts_hbm,
                    idx2_v, rows_v, zeros_v, ones_v, shared_counts, sem):
    c = lax.axis_index("c")
    s = lax.axis_index("s")
    wid = s * 2 + c
    base = wid * _BPW
    pltpu.sync_copy(idx3_hbm.at[wid], idx2_v)
    for j in range(_N128):
        pltpu.async_copy(table_hbm.at[idx2_v.at[j]],
                         rows_v.at[pl.ds(j * 128, 128)], sem)
    for j in range(_N128):
        pltpu.make_async_copy(table_hbm.at[idx2_v.at[j]],
                              rows_v.at[pl.ds(j * 128, 128)], sem).wait()
    pltpu.sync_copy(rows_v, q_hbm.at[pl.ds(base, _BPW)])

    zeros = jnp.zeros((16,), jnp.float32)

    def _clear(j, _):
        zeros_v[pl.ds(j * 16, 16)] = zeros
        return 0

    ones = jnp.ones((16,), jnp.float32)
    for j in range(128 // 16):
        ones_v[pl.ds(j * 16, 16)] = ones

    @pl.when(s == 0)
    def _():
        lax.fori_loop(0, _NUM_CODES // 16, _clear, 0)
        pltpu.sync_copy(zeros_v, shared_counts)

    plsc.subcore_barrier()
    for j in range(_N128):
        pltpu.sync_copy(ones_v, shared_counts.at[idx2_v.at[j]], add=True)
    plsc.subcore_barrier()

    @pl.when(s == 0)
    def _():
        pltpu.sync_copy(shared_counts, counts_hbm.at[c])


def _stats_body(counts_ref, ent_ref, used_ref):
    counts = jnp.sum(counts_ref[...], axis=0, keepdims=True)
    usage = counts * (1.0 / _B) + 1e-10
    ent_ref[...] = jnp.reshape(-jnp.sum(usage * jnp.log(usage)), (1, 1))
    used_ref[...] = jnp.reshape(
        jnp.sum((counts > 0.0).astype(jnp.int32)), (1, 1))


def _tc_stats(partials):
    return pl.pallas_call(
        _stats_body,
        out_shape=[
            jax.ShapeDtypeStruct((1, 1), jnp.float32),
            jax.ShapeDtypeStruct((1, 1), jnp.int32),
        ],
    )(partials)


def kernel(z, codebook):
    zs = (-2.0 * z).astype(jnp.bfloat16)
    cbb = codebook.astype(jnp.bfloat16)
    cbb32 = cbb.astype(jnp.float32)
    table = jnp.pad(cbb32, ((0, 0), (0, _PD - _DIM)))
    idx, idx3, commit = _tc_main(z, zs, codebook, cbb)
    q_pad, partials = _sc_gather_hist(table, idx3)
    ent, used = _tc_stats(partials)
    q = q_pad[:, :_DIM]
    return (q, idx, commit[0, 0], ent[0, 0], used[0, 0])


# speedup vs baseline: 1.3447x; 1.1305x over previous
import functools

import jax
import jax.numpy as jnp
from jax import lax
from jax.experimental import pallas as pl
from jax.experimental.pallas import tpu as pltpu
from jax.experimental.pallas import tpu_sc as plsc

_NUM_CODES = 8192
_DIM = 32
_BETA = 0.25
_B = 8192
_BB = 512
_GRID = _B // _BB
_NWIN = 4
_W = _NUM_CODES // _NWIN

_NW = 32
_BPW = _B // _NW
_PD = 128


def _vq_body(z_ref, zs_ref, cb_ref, cbb_ref, idx3_ref, commit_ref,
             e2_ref, acc_ref):
    i = pl.program_id(0)
    z = z_ref[...]
    zs = zs_ref[...]
    cbb = cbb_ref[...]

    @pl.when(i == 0)
    def _():
        cb = cb_ref[...]
        e2_ref[...] = jnp.sum(cb * cb, axis=1)[None, :]

    z2 = jnp.sum(z * z, axis=1, keepdims=True)
    e2 = e2_ref[...]
    nze2 = jax.lax.dot_general(zs, cbb, (((1,), (1,)), ((), ())),
                               preferred_element_type=jnp.float32)
    dist = (z2 + e2) + nze2

    iota_k = jax.lax.broadcasted_iota(jnp.int32, (1, _NUM_CODES), 1)

    acc_v = jnp.full((_BB, 1), jnp.inf, jnp.float32)
    acc_i = jnp.zeros((_BB, 1), jnp.int32)
    tv = jnp.full((_BB, 1), jnp.inf, jnp.float32)
    for w in range(_NWIN):
        dw = jax.lax.slice_in_dim(dist, w * _W, (w + 1) * _W, axis=1)
        io = jax.lax.slice_in_dim(iota_k, w * _W, (w + 1) * _W, axis=1)
        wmin = jnp.min(dw, axis=1, keepdims=True)
        widx = jnp.min(jnp.where(dw == wmin, io, _NUM_CODES), axis=1,
                       keepdims=True)
        lt = wmin < acc_v
        take = lt | ((wmin == acc_v) & (widx < acc_i))
        acc_v = jnp.where(lt, wmin, acc_v)
        acc_v = acc_v.astype(jnp.bfloat16).astype(jnp.float32)
        acc_i = jnp.where(take, widx, acc_i)
        tv = jnp.where(take, wmin, tv)
    idx3_ref[...] = jnp.reshape(acc_i[:, 0], (_BB // 256, 2, 128))

    block_commit = jnp.sum(tv)

    @pl.when(i == 0)
    def _():
        acc_ref[0] = block_commit

    @pl.when(i > 0)
    def _():
        acc_ref[0] += block_commit

    @pl.when(i == _GRID - 1)
    def _():
        commit_ref[...] = jnp.reshape(
            acc_ref[0] * (_BETA / (_B * _DIM)), (1, 1))


def _tc_main(z, zs, codebook, cbb):
    return pl.pallas_call(
        _vq_body,
        grid=(_GRID,),
        in_specs=[
            pl.BlockSpec((_BB, _DIM), lambda i: (i, 0)),
            pl.BlockSpec((_BB, _DIM), lambda i: (i, 0)),
            pl.BlockSpec((_NUM_CODES, _DIM), lambda i: (0, 0)),
            pl.BlockSpec((_NUM_CODES, _DIM), lambda i: (0, 0)),
        ],
        out_specs=[
            pl.BlockSpec((_BB // 256, 2, 128), lambda i: (i, 0, 0)),
            pl.BlockSpec((1, 1), lambda i: (0, 0)),
        ],
        out_shape=[
            jax.ShapeDtypeStruct((_NW, 2, 128), jnp.int32),
            jax.ShapeDtypeStruct((1, 1), jnp.float32),
        ],
        scratch_shapes=[
            pltpu.VMEM((1, _NUM_CODES), jnp.float32),
            pltpu.SMEM((1,), jnp.float32),
        ],
    )(z, zs, codebook, cbb)


_N128 = _BPW // 128
_CLR = _NUM_CODES // 16


@functools.partial(
    pl.kernel,
    out_type=[
        jax.ShapeDtypeStruct((_B, _PD), jnp.float32),
        jax.ShapeDtypeStruct((2, _NUM_CODES), jnp.float32),
    ],
    mesh=plsc.VectorSubcoreMesh(core_axis_name="c", subcore_axis_name="s"),
    scratch_types=[
        pltpu.VMEM((_N128, 128), jnp.int32),
        pltpu.VMEM((_BPW, _PD), jnp.float32),
        pltpu.VMEM((_CLR,), jnp.float32),
        pltpu.VMEM((128,), jnp.float32),
        pltpu.VMEM_SHARED((_NUM_CODES,), jnp.float32),
        pltpu.SemaphoreType.DMA,
    ],
)
def _sc_gather_hist(table_hbm, idx3_hbm, q_hbm, counts_hbm,
                    idx2_v, rows_v, zeros_v, ones_v, shared_counts, sem):
    c = lax.axis_index("c")
    s = lax.axis_index("s")
    wid = s * 2 + c
    base = wid * _BPW
    pltpu.sync_copy(idx3_hbm.at[wid], idx2_v)
    for j in range(_N128):
        pltpu.async_copy(table_hbm.at[idx2_v.at[j]],
                         rows_v.at[pl.ds(j * 128, 128)], sem)

    zeros = jnp.zeros((16,), jnp.float32)
    for j in range(_CLR // 16):
        zeros_v[pl.ds(j * 16, 16)] = zeros
    ones = jnp.ones((16,), jnp.float32)
    for j in range(128 // 16):
        ones_v[pl.ds(j * 16, 16)] = ones
    pltpu.sync_copy(zeros_v, shared_counts.at[pl.ds(s * _CLR, _CLR)])
    plsc.subcore_barrier()
    for j in range(_N128):
        pltpu.sync_copy(ones_v, shared_counts.at[idx2_v.at[j]], add=True)
    plsc.subcore_barrier()

    @pl.when(s == 0)
    def _():
        pltpu.sync_copy(shared_counts, counts_hbm.at[c])

    for j in range(_N128):
        pltpu.make_async_copy(table_hbm.at[idx2_v.at[j]],
                              rows_v.at[pl.ds(j * 128, 128)], sem).wait()
    pltpu.sync_copy(rows_v, q_hbm.at[pl.ds(base, _BPW)])


def _stats_body(counts_ref, ent_ref, used_ref):
    counts = jnp.sum(counts_ref[...], axis=0, keepdims=True)
    usage = counts * (1.0 / _B) + 1e-10
    ent_ref[...] = jnp.reshape(-jnp.sum(usage * jnp.log(usage)), (1, 1))
    used_ref[...] = jnp.reshape(
        jnp.sum((counts > 0.0).astype(jnp.int32)), (1, 1))


def _tc_stats(partials):
    return pl.pallas_call(
        _stats_body,
        out_shape=[
            jax.ShapeDtypeStruct((1, 1), jnp.float32),
            jax.ShapeDtypeStruct((1, 1), jnp.int32),
        ],
    )(partials)


def kernel(z, codebook):
    zs = (-2.0 * z).astype(jnp.bfloat16)
    cbb = codebook.astype(jnp.bfloat16)
    cbb32 = cbb.astype(jnp.float32)
    table = jnp.pad(cbb32, ((0, 0), (0, _PD - _DIM)))
    idx3, commit = _tc_main(z, zs, codebook, cbb)
    q_pad, partials = _sc_gather_hist(table, idx3)
    ent, used = _tc_stats(partials)
    q = q_pad[:, :_DIM]
    idx = idx3.reshape(_B)
    return (q, idx, commit[0, 0], ent[0, 0], used[0, 0])
